# baseline (speedup 1.0000x reference)
import functools

import jax
import jax.numpy as jnp
from jax import lax
from jax.experimental import pallas as pl
from jax.experimental.pallas import tpu as pltpu
from jax.experimental.pallas import tpu_sc as plsc

N_TASKS = 8
N_EXPERTS = 16
TOP_K = 2
W_EX = 0.01
TOKENS = 8192

NC = 2
NS = 16
NW = NC * NS
ROWS = TOKENS // NW
GROUPS = ROWS // 16
STRIDE_L = N_EXPERTS + 1
STRIDE = 16 * STRIDE_L


def _sc_body(logits_hbm, labels_hbm, out_hbm, logits_v, labels_v, acc_v, seg_v,
             sem):
    wid = lax.axis_index("s") * NC + lax.axis_index("c")
    base = wid * ROWS
    cp1 = pltpu.async_copy(logits_hbm.at[:, pl.ds(base, ROWS)], logits_v, sem)
    cp2 = pltpu.async_copy(labels_hbm.at[pl.ds(base, ROWS)], labels_v, sem)

    zero = jnp.zeros((16,), jnp.float32)

    @plsc.parallel_loop(0, N_TASKS * STRIDE // 16, step=8)
    def zinit(i):
        for k in range(8):
            acc_v[pl.ds((i + k) * 16, 16)] = zero

    cp1.wait()
    cp2.wait()

    lane = lax.iota(jnp.int32, 16)
    lane17 = lane * STRIDE_L

    @plsc.parallel_loop(0, GROUPS, unroll=2)
    def group(j):
        exps = [jnp.exp(logits_v[e, pl.ds(j * 16, 16)])
                for e in range(N_EXPERTS)]
        t = exps
        while len(t) > 1:
            t = [t[i] + t[i + 1] for i in range(0, len(t), 2)]
        r = 1.0 / t[0]
        base = labels_v[pl.ds(j * 16, 16)] * STRIDE + lane17
        for e in range(N_EXPERTS):
            plsc.addupdate_scatter(acc_v, [base + e], exps[e] * r)

    def red(l, tots):
        off = l * STRIDE_L
        return tuple(
            tots[t] + plsc.load_gather(acc_v, [lane + (t * STRIDE) + off])
            for t in range(N_TASKS))

    segs = lax.fori_loop(0, 16, red, (zero,) * N_TASKS)
    for t in range(N_TASKS):
        seg_v[t] = segs[t]
    pltpu.sync_copy(seg_v, out_hbm.at[wid])


_sc_partials = functools.partial(
    pl.kernel,
    out_type=jax.ShapeDtypeStruct((NW, N_TASKS, N_EXPERTS), jnp.float32),
    mesh=plsc.VectorSubcoreMesh(core_axis_name="c", subcore_axis_name="s",
                                num_cores=NC),
    compiler_params=pltpu.CompilerParams(needs_layout_passes=False),
    scratch_types=[
        pltpu.VMEM((N_EXPERTS, ROWS), jnp.float32),
        pltpu.VMEM((ROWS,), jnp.int32),
        pltpu.VMEM((N_TASKS * STRIDE,), jnp.float32),
        pltpu.VMEM((N_TASKS, N_EXPERTS), jnp.float32),
        pltpu.SemaphoreType.DMA,
    ],
)(_sc_body)


def _tc_body(part_ref, lab_ref, out_ref):
    seg = jnp.zeros((N_TASKS, N_EXPERTS), jnp.float32)
    for k in range(NW):
        seg = seg + part_ref[pl.ds(k * N_TASKS, N_TASKS), :]
    lab = lab_ref[...]
    rowid = lax.broadcasted_iota(jnp.int32, (N_TASKS, N_EXPERTS), 0)
    gate = jnp.zeros((N_TASKS, N_EXPERTS), jnp.float32)
    for t in range(N_TASKS):
        ct = jnp.sum((lab == t).astype(jnp.float32))
        gate = gate + jnp.where(rowid == t, ct, 0.0)
    ex_gate = gate * seg
    tot = jnp.sum(ex_gate) / TOP_K
    ex = ex_gate / (tot + 0.0001)
    p_ti = jnp.sum(ex, axis=1, keepdims=True) + 0.0001
    p_ei = jnp.sum(ex, axis=0, keepdims=True) + 0.0001
    expert_loss = -jnp.sum(ex * jnp.log(ex / p_ti / p_ei + 0.0001))
    out_ref[0, 0] = W_EX * expert_loss


def _tc_loss(partials, labels2d):
    return pl.pallas_call(
        _tc_body,
        out_shape=jax.ShapeDtypeStruct((1, 1), jnp.float32),
        out_specs=pl.BlockSpec(memory_space=pltpu.SMEM),
    )(partials, labels2d)


def kernel(router_logits, router_labels):
    logits = lax.stop_gradient(router_logits.astype(jnp.float32))
    labels = router_labels.astype(jnp.int32)
    partials = _sc_partials(logits.T, labels)
    loss = _tc_loss(partials.reshape(NW * N_TASKS, N_EXPERTS),
                    labels.reshape(64, 128))
    return loss.reshape(())


# speedup vs baseline: 1.0412x; 1.0412x over previous
import functools

import jax
import jax.numpy as jnp
from jax import lax
from jax.experimental import pallas as pl
from jax.experimental.pallas import tpu as pltpu
from jax.experimental.pallas import tpu_sc as plsc

N_TASKS = 8
N_EXPERTS = 16
TOP_K = 2
W_EX = 0.01
TOKENS = 8192

NC = 2
NS = 16
NW = NC * NS
ROWS = TOKENS // NW
GROUPS = ROWS // 16
STRIDE_L = N_EXPERTS + 1
STRIDE = 16 * STRIDE_L


def _sc_body(logits_hbm, labels_hbm, out_hbm, logits_v, labels_v, acc_v, seg_v,
             sem):
    wid = lax.axis_index("s") * NC + lax.axis_index("c")
    base = wid * ROWS
    cp1 = pltpu.async_copy(logits_hbm.at[:, pl.ds(base, ROWS)], logits_v, sem)
    cp2 = pltpu.async_copy(labels_hbm.at[pl.ds(base, ROWS)], labels_v, sem)

    zero = jnp.zeros((16,), jnp.float32)

    @plsc.parallel_loop(0, N_TASKS * STRIDE // 16, step=8)
    def zinit(i):
        for k in range(8):
            acc_v[pl.ds((i + k) * 16, 16)] = zero

    cp1.wait()
    cp2.wait()

    lane = lax.iota(jnp.int32, 16)
    lane17 = lane * STRIDE_L

    @plsc.parallel_loop(0, GROUPS, unroll=1)
    def group(j):
        exps = [jnp.exp(logits_v[e, pl.ds(j * 16, 16)])
                for e in range(N_EXPERTS)]
        t = exps
        while len(t) > 1:
            t = [t[i] + t[i + 1] for i in range(0, len(t), 2)]
        r = 1.0 / t[0]
        base = labels_v[pl.ds(j * 16, 16)] * STRIDE + lane17
        for e in range(N_EXPERTS):
            plsc.addupdate_scatter(acc_v, [base + e], exps[e] * r)

    def red(l, tots):
        off = l * STRIDE_L
        return tuple(
            tots[t] + plsc.load_gather(acc_v, [lane + (t * STRIDE) + off])
            for t in range(N_TASKS))

    segs = lax.fori_loop(0, 16, red, (zero,) * N_TASKS)
    for t in range(N_TASKS):
        seg_v[t] = segs[t]
    pltpu.sync_copy(seg_v, out_hbm.at[wid])


_sc_partials = functools.partial(
    pl.kernel,
    out_type=jax.ShapeDtypeStruct((NW, N_TASKS, N_EXPERTS), jnp.float32),
    mesh=plsc.VectorSubcoreMesh(core_axis_name="c", subcore_axis_name="s",
                                num_cores=NC),
    compiler_params=pltpu.CompilerParams(needs_layout_passes=False),
    scratch_types=[
        pltpu.VMEM((N_EXPERTS, ROWS), jnp.float32),
        pltpu.VMEM((ROWS,), jnp.int32),
        pltpu.VMEM((N_TASKS * STRIDE,), jnp.float32),
        pltpu.VMEM((N_TASKS, N_EXPERTS), jnp.float32),
        pltpu.SemaphoreType.DMA,
    ],
)(_sc_body)


def _tc_body(part_ref, lab_ref, out_ref):
    seg = jnp.zeros((N_TASKS, N_EXPERTS), jnp.float32)
    for k in range(NW):
        seg = seg + part_ref[pl.ds(k * N_TASKS, N_TASKS), :]
    lab = lab_ref[...]
    rowid = lax.broadcasted_iota(jnp.int32, (N_TASKS, N_EXPERTS), 0)
    gate = jnp.zeros((N_TASKS, N_EXPERTS), jnp.float32)
    for t in range(N_TASKS):
        ct = jnp.sum((lab == t).astype(jnp.float32))
        gate = gate + jnp.where(rowid == t, ct, 0.0)
    ex_gate = gate * seg
    tot = jnp.sum(ex_gate) / TOP_K
    ex = ex_gate / (tot + 0.0001)
    p_ti = jnp.sum(ex, axis=1, keepdims=True) + 0.0001
    p_ei = jnp.sum(ex, axis=0, keepdims=True) + 0.0001
    expert_loss = -jnp.sum(ex * jnp.log(ex / p_ti / p_ei + 0.0001))
    out_ref[0, 0] = W_EX * expert_loss


def _tc_loss(partials, labels2d):
    return pl.pallas_call(
        _tc_body,
        out_shape=jax.ShapeDtypeStruct((1, 1), jnp.float32),
        out_specs=pl.BlockSpec(memory_space=pltpu.SMEM),
    )(partials, labels2d)


def kernel(router_logits, router_labels):
    logits = lax.stop_gradient(router_logits.astype(jnp.float32))
    labels = router_labels.astype(jnp.int32)
    partials = _sc_partials(logits.T, labels)
    loss = _tc_loss(partials.reshape(NW * N_TASKS, N_EXPERTS),
                    labels.reshape(64, 128))
    return loss.reshape(())


# speedup vs baseline: 1.0486x; 1.0071x over previous
import functools

import jax
import jax.numpy as jnp
from jax import lax
from jax.experimental import pallas as pl
from jax.experimental.pallas import tpu as pltpu
from jax.experimental.pallas import tpu_sc as plsc

N_TASKS = 8
N_EXPERTS = 16
TOP_K = 2
W_EX = 0.01
TOKENS = 8192

NC = 2
NS = 16
NW = NC * NS
ROWS = TOKENS // NW
GROUPS = ROWS // 16
STRIDE_L = N_EXPERTS + 1
STRIDE = 16 * STRIDE_L


def _sc_body(logits_hbm, labels_hbm, out_hbm, logits_v, labels_v, acc_v, seg_v,
             sem):
    wid = lax.axis_index("s") * NC + lax.axis_index("c")
    base = wid * ROWS
    cp1 = pltpu.async_copy(logits_hbm.at[:, pl.ds(base, ROWS)], logits_v, sem)
    cp2 = pltpu.async_copy(labels_hbm.at[pl.ds(base, ROWS)], labels_v, sem)

    zero = jnp.zeros((16,), jnp.float32)

    @plsc.parallel_loop(0, N_TASKS * STRIDE // 16, step=8)
    def zinit(i):
        for k in range(8):
            acc_v[pl.ds((i + k) * 16, 16)] = zero

    cp1.wait()
    cp2.wait()

    lane = lax.iota(jnp.int32, 16)
    lane17 = lane * STRIDE_L

    @plsc.parallel_loop(0, GROUPS, unroll=1)
    def group(j):
        exps = [jnp.exp(logits_v[e, pl.ds(j * 16, 16)])
                for e in range(N_EXPERTS)]
        t = exps
        while len(t) > 1:
            t = [t[i] + t[i + 1] for i in range(0, len(t), 2)]
        r = 1.0 / t[0]
        base = labels_v[pl.ds(j * 16, 16)] * STRIDE + lane17
        for e in range(N_EXPERTS):
            plsc.addupdate_scatter(acc_v, [base + e], exps[e] * r)

    def red(l, tots):
        off = l * STRIDE_L
        return tuple(
            tots[t] + plsc.load_gather(acc_v, [lane + (t * STRIDE) + off])
            for t in range(N_TASKS))

    segs = lax.fori_loop(0, 16, red, (zero,) * N_TASKS)
    for t in range(N_TASKS):
        seg_v[t] = segs[t]
    pltpu.sync_copy(seg_v, out_hbm.at[wid])


_sc_partials = functools.partial(
    pl.kernel,
    out_type=jax.ShapeDtypeStruct((NW, N_TASKS, N_EXPERTS), jnp.float32),
    mesh=plsc.VectorSubcoreMesh(core_axis_name="c", subcore_axis_name="s",
                                num_cores=NC),
    compiler_params=pltpu.CompilerParams(needs_layout_passes=False),
    scratch_types=[
        pltpu.VMEM((N_EXPERTS, ROWS), jnp.float32),
        pltpu.VMEM((ROWS,), jnp.int32),
        pltpu.VMEM((N_TASKS * STRIDE,), jnp.float32),
        pltpu.VMEM((N_TASKS, N_EXPERTS), jnp.float32),
        pltpu.SemaphoreType.DMA,
    ],
)(_sc_body)


def _counts_body(lab_ref, gate_ref):
    lab = lab_ref[...]
    rowid = lax.broadcasted_iota(jnp.int32, (N_TASKS, N_EXPERTS), 0)
    gate = jnp.zeros((N_TASKS, N_EXPERTS), jnp.float32)
    for t in range(N_TASKS):
        ct = jnp.sum((lab == t).astype(jnp.float32))
        gate = gate + jnp.where(rowid == t, ct, 0.0)
    gate_ref[...] = gate


def _tc_counts(labels2d):
    return pl.pallas_call(
        _counts_body,
        out_shape=jax.ShapeDtypeStruct((N_TASKS, N_EXPERTS), jnp.float32),
    )(labels2d)


def _tc_body(part_ref, gate_ref, out_ref):
    seg = jnp.zeros((N_TASKS, N_EXPERTS), jnp.float32)
    for k in range(NW):
        seg = seg + part_ref[pl.ds(k * N_TASKS, N_TASKS), :]
    ex_gate = gate_ref[...] * seg
    tot = jnp.sum(ex_gate) / TOP_K
    ex = ex_gate / (tot + 0.0001)
    p_ti = jnp.sum(ex, axis=1, keepdims=True) + 0.0001
    p_ei = jnp.sum(ex, axis=0, keepdims=True) + 0.0001
    expert_loss = -jnp.sum(ex * jnp.log(ex / p_ti / p_ei + 0.0001))
    out_ref[0, 0] = W_EX * expert_loss


def _tc_loss(partials, gate):
    return pl.pallas_call(
        _tc_body,
        out_shape=jax.ShapeDtypeStruct((1, 1), jnp.float32),
        out_specs=pl.BlockSpec(memory_space=pltpu.SMEM),
    )(partials, gate)


def kernel(router_logits, router_labels):
    logits = lax.stop_gradient(router_logits.astype(jnp.float32))
    labels = router_labels.astype(jnp.int32)
    partials = _sc_partials(logits.T, labels)
    gate = _tc_counts(labels.reshape(64, 128))
    loss = _tc_loss(partials.reshape(NW * N_TASKS, N_EXPERTS), gate)
    return loss.reshape(())
